# Initial kernel scaffold; baseline (speedup 1.0000x reference)
import jax
import jax.numpy as jnp
from jax.experimental import pallas as pl


def kernel(node_feature, edge_index, edge_feature, H, Wm_xz, Ws_xz, b_xz, Wm_hz, Ws_hz, b_hz, Wm_xr, Ws_xr, b_xr, Wm_hr, Ws_hr, b_hr, Wm_xh, Ws_xh, b_xh, Wm_hh, Ws_hh, b_hh):
    raise NotImplementedError("write your pallas kernel here")


# speedup vs baseline: 3.8206x; 3.8206x over previous
import functools

import jax
import jax.numpy as jnp
from jax import lax
from jax.experimental import pallas as pl
from jax.experimental.pallas import tpu as pltpu
from jax.experimental.pallas import tpu_sc as plsc

N = 10000
D = 128
DE = 16
E = 320000
CH = 128
CT = 2528
EPAD = CT * CH
NACC = 10112
ZROWS = NACC // 16
CPT1 = CT // 16
HCPT = CPT1 // 2
CPT2 = CT // 32

_f32 = jnp.float32


def _sc_pass1(xh_hbm, ef_hbm, src2_hbm, dst_hbm, z128_hbm,
              sxh_out, se_out,
              acc128, src_v, dst_v, rows_v, sem, sem2):
    c = lax.axis_index("c")
    s = lax.axis_index("s")
    r0 = s * ZROWS
    pltpu.sync_copy(z128_hbm.at[pl.ds(r0, ZROWS)], acc128.at[pl.ds(r0, ZROWS)])
    plsc.subcore_barrier()

    base = s * CPT1
    iofs = c * EPAD

    def body(j, carry):
        row = base + j
        pltpu.sync_copy(src2_hbm.at[pl.ds(iofs + row * CH, CH)], src_v)
        pltpu.sync_copy(dst_hbm.at[pl.ds(row * CH, CH)], dst_v)
        pltpu.async_copy(xh_hbm.at[src_v], rows_v, sem).wait()
        pltpu.async_copy(rows_v, acc128.at[dst_v], sem2, add=True).wait()
        return carry

    lax.fori_loop(0, CPT1, body, 0)
    plsc.subcore_barrier()
    pltpu.sync_copy(acc128.at[pl.ds(r0, ZROWS)], sxh_out.at[c, pl.ds(r0, ZROWS)])

    pltpu.sync_copy(z128_hbm.at[pl.ds(r0, ZROWS)], acc128.at[pl.ds(r0, ZROWS)])
    plsc.subcore_barrier()

    ebase = (c * 16 + s) * CPT2

    def body_e(j, carry):
        row = ebase + j
        pltpu.sync_copy(dst_hbm.at[pl.ds(row * CH, CH)], dst_v)
        pltpu.sync_copy(ef_hbm.at[pl.ds(row * CH, CH)], rows_v)
        pltpu.async_copy(rows_v, acc128.at[dst_v], sem2, add=True).wait()
        return carry

    lax.fori_loop(0, CPT2, body_e, 0)
    plsc.subcore_barrier()
    pltpu.sync_copy(acc128.at[pl.ds(r0, ZROWS)], se_out.at[c, pl.ds(r0, ZROWS)])


def _sc_pass2(g_hbm, src2_hbm, dst_hbm, z128_hbm,
              sg_out,
              acc128, src_v, dst_v, rows_v, sem, sem2):
    c = lax.axis_index("c")
    s = lax.axis_index("s")
    r0 = s * ZROWS
    pltpu.sync_copy(z128_hbm.at[pl.ds(r0, ZROWS)], acc128.at[pl.ds(r0, ZROWS)])
    plsc.subcore_barrier()

    base = (c * 16 + s) * CPT2

    def body(j, carry):
        row = base + j
        pltpu.sync_copy(src2_hbm.at[pl.ds(row * CH, CH)], src_v)
        pltpu.sync_copy(dst_hbm.at[pl.ds(row * CH, CH)], dst_v)
        pltpu.async_copy(g_hbm.at[src_v], rows_v, sem).wait()
        pltpu.async_copy(rows_v, acc128.at[dst_v], sem2, add=True).wait()
        return carry

    lax.fori_loop(0, CPT2, body, 0)
    plsc.subcore_barrier()

    pltpu.sync_copy(acc128.at[pl.ds(r0, ZROWS)], sg_out.at[c, pl.ds(r0, ZROWS)])


@functools.lru_cache(maxsize=1)
def _build_sc_kernels():
    mesh = plsc.VectorSubcoreMesh(core_axis_name="c", subcore_axis_name="s",
                                  num_cores=2, num_subcores=16)
    pass1 = functools.partial(
        pl.kernel,
        out_type=(
            jax.ShapeDtypeStruct((2, NACC, D), _f32),
            jax.ShapeDtypeStruct((2, NACC, D), _f32),
        ),
        mesh=mesh,
        scratch_types=[
            pltpu.VMEM_SHARED((NACC, D), _f32),
            pltpu.VMEM((CH,), jnp.int32),
            pltpu.VMEM((CH,), jnp.int32),
            pltpu.VMEM((CH, D), _f32),
            pltpu.SemaphoreType.DMA,
            pltpu.SemaphoreType.DMA,
        ],
    )(_sc_pass1)
    pass2 = functools.partial(
        pl.kernel,
        out_type=jax.ShapeDtypeStruct((2, NACC, D), _f32),
        mesh=mesh,
        scratch_types=[
            pltpu.VMEM_SHARED((NACC, D), _f32),
            pltpu.VMEM((CH,), jnp.int32),
            pltpu.VMEM((CH,), jnp.int32),
            pltpu.VMEM((CH, D), _f32),
            pltpu.SemaphoreType.DMA,
            pltpu.SemaphoreType.DMA,
        ],
    )(_sc_pass2)
    return pass1, pass2


BR = 400
GRID = N // BR


def _tc1_body(x, h, sx, sh, sea, seb, w1, w3, w5, w2, w4, wshh, bv,
              z_o, g_o, t2_o):
    se = sea[...] + seb[...]
    u = (jnp.dot(x[...], w1[...], preferred_element_type=_f32)
         + jnp.dot(sx[...], w3[...], preferred_element_type=_f32)
         + jnp.dot(se, w5[...], preferred_element_type=_f32)
         + bv[...])
    u2 = (u[:, 0:2 * D]
          + jnp.dot(h[...], w2[...], preferred_element_type=_f32)
          + jnp.dot(sh[...], w4[...], preferred_element_type=_f32))
    z = jax.nn.sigmoid(u2[:, 0:D])
    r = jax.nn.sigmoid(u2[:, D:2 * D])
    g = h[...] * r
    z_o[...] = z
    g_o[...] = g
    t2_o[...] = u[:, 2 * D:3 * D] + jnp.dot(g, wshh[...], preferred_element_type=_f32)


_row_spec = pl.BlockSpec((BR, D), lambda i: (i, 0))

_tc1 = pl.pallas_call(
    _tc1_body,
    grid=(GRID,),
    in_specs=[
        _row_spec,
        _row_spec,
        _row_spec,
        _row_spec,
        _row_spec,
        _row_spec,
        pl.BlockSpec((D, 3 * D), lambda i: (0, 0)),
        pl.BlockSpec((D, 3 * D), lambda i: (0, 0)),
        pl.BlockSpec((D, 3 * D), lambda i: (0, 0)),
        pl.BlockSpec((D, 2 * D), lambda i: (0, 0)),
        pl.BlockSpec((D, 2 * D), lambda i: (0, 0)),
        pl.BlockSpec((D, D), lambda i: (0, 0)),
        pl.BlockSpec((1, 3 * D), lambda i: (0, 0)),
    ],
    out_specs=[_row_spec, _row_spec, _row_spec],
    out_shape=[
        jax.ShapeDtypeStruct((N, D), _f32),
        jax.ShapeDtypeStruct((N, D), _f32),
        jax.ShapeDtypeStruct((N, D), _f32),
    ],
)


def _tc2_body(z, h, t2, sga, sgb, wmhh, out):
    sg = sga[...] + sgb[...]
    ht = jnp.tanh(t2[...] + jnp.dot(sg, wmhh[...], preferred_element_type=_f32))
    out[...] = z[...] * h[...] + (1.0 - z[...]) * ht


_tc2 = pl.pallas_call(
    _tc2_body,
    grid=(GRID,),
    in_specs=[
        _row_spec,
        _row_spec,
        _row_spec,
        _row_spec,
        _row_spec,
        pl.BlockSpec((D, D), lambda i: (0, 0)),
    ],
    out_specs=_row_spec,
    out_shape=jax.ShapeDtypeStruct((N, D), _f32),
)


def kernel(node_feature, edge_index, edge_feature, H,
           Wm_xz, Ws_xz, b_xz, Wm_hz, Ws_hz, b_hz,
           Wm_xr, Ws_xr, b_xr, Wm_hr, Ws_hr, b_hr,
           Wm_xh, Ws_xh, b_xh, Wm_hh, Ws_hh, b_hh):
    src = edge_index[0]
    dst = edge_index[1]
    pad = EPAD - E
    srcp = jnp.concatenate([src, jnp.zeros((pad,), jnp.int32)])
    src2 = jnp.concatenate([srcp, srcp + N])
    dstp = jnp.concatenate([dst, jnp.full((pad,), N, jnp.int32)])
    efp = jnp.pad(edge_feature, ((0, pad), (0, D - DE)))
    xh = jnp.concatenate([node_feature, H], axis=0)
    z128 = jnp.zeros((NACC, D), _f32)

    sc1, sc2 = _build_sc_kernels()
    sxh, se2 = sc1(xh, efp, src2, dstp, z128)
    sx = sxh[0]
    sh = sxh[1]

    w1 = jnp.concatenate([Ws_xz, Ws_xr, Ws_xh], axis=1)
    w3 = jnp.concatenate([Wm_xz[:D], Wm_xr[:D], Wm_xh[:D]], axis=1)
    w5 = jnp.concatenate([Wm_xz[D:] + Wm_hz[D:],
                          Wm_xr[D:] + Wm_hr[D:],
                          Wm_xh[D:] + Wm_hh[D:]], axis=1)
    w5 = jnp.pad(w5, ((0, D - DE), (0, 0)))
    w2 = jnp.concatenate([Ws_hz, Ws_hr], axis=1)
    w4 = jnp.concatenate([Wm_hz[:D], Wm_hr[:D]], axis=1)
    bv = jnp.concatenate([b_xz + b_hz, b_xr + b_hr, b_xh + b_hh]).reshape(1, 3 * D)

    z, g, t2 = _tc1(node_feature, H, sx, sh, se2[0], se2[1],
                    w1, w3, w5, w2, w4, Ws_hh, bv)

    sg2 = sc2(g, src2, dstp, z128)

    return _tc2(z, H, t2, sg2[0], sg2[1], Wm_hh[:D])
